# baseline (speedup 1.0000x reference)
import functools

import jax
import jax.numpy as jnp
from jax import lax
from jax.experimental import pallas as pl
from jax.experimental.pallas import tpu as pltpu
from jax.experimental.pallas import tpu_sc as plsc

VOCAB = 1000
D = 50
DP = 64
CHARS = 20
NUM_CORES = 2
NUM_SUBCORES = 16
NTILES = NUM_CORES * NUM_SUBCORES
CHUNK = 640
GROUP = 8


def _build_sc_call(n_words):
    words_per_tile = n_words // NTILES
    n_chunks = words_per_tile // CHUNK
    n_pairs = n_chunks // 2
    mesh = plsc.VectorSubcoreMesh(core_axis_name="c", subcore_axis_name="s")

    @functools.partial(
        pl.kernel,
        out_type=jax.ShapeDtypeStruct((n_words * D,), jnp.float32),
        mesh=mesh,
        compiler_params=pltpu.CompilerParams(needs_layout_passes=False),
        scratch_types=[
            pltpu.VMEM((VOCAB * DP // 2,), jnp.int32),
            pltpu.VMEM((CHUNK * CHARS,), jnp.int32),
            pltpu.VMEM((CHUNK * CHARS,), jnp.int32),
            pltpu.VMEM((CHUNK * D + 16,), jnp.float32),
            pltpu.VMEM((CHUNK * D + 16,), jnp.float32),
            pltpu.SemaphoreType.DMA,
            pltpu.SemaphoreType.DMA,
            pltpu.SemaphoreType.DMA,
            pltpu.SemaphoreType.DMA,
        ],
    )
    def sc_kernel(
        table_hbm, x_hbm, out_hbm,
        table_v, idx0, idx1, out0, out1, si0, si1, so0, so1,
    ):
        wid = lax.axis_index("s") * NUM_CORES + lax.axis_index("c")
        tile_base = wid * words_per_tile
        idxb, outb = (idx0, idx1), (out0, out1)
        sin, son = (si0, si1), (so0, so1)

        def in_slice(ci):
            return x_hbm.at[
                pl.ds((tile_base + ci * CHUNK) * CHARS, CHUNK * CHARS)
            ]

        def out_slice(ci):
            return out_hbm.at[pl.ds((tile_base + ci * CHUNK) * D, CHUNK * D)]

        pltpu.async_copy(in_slice(0), idx0, si0)
        pltpu.sync_copy(table_hbm, table_v)

        def compute_chunk(idx_v, out_v):
            def group_body(g, carry2):
                base = g * (GROUP * CHARS)
                iv = [
                    idx_v[pl.ds(base + 16 * j, 16)] * (DP // 2)
                    for j in range(GROUP * CHARS // 16)
                ]
                for w in range(GROUP):
                    acc = [jnp.zeros((16,), jnp.float32) for _ in range(4)]
                    ac2 = [jnp.zeros((16,), jnp.float32) for _ in range(4)]
                    banks = (acc, ac2)
                    for cp in range(CHARS // 2):
                        f0 = w * CHARS + 2 * cp
                        i0 = iv[f0 // 16][f0 % 16]
                        i1 = iv[(f0 + 1) // 16][(f0 + 1) % 16]
                        for h in range(2):
                            v0 = table_v[pl.ds(i0 + h * 16, 16)]
                            v1 = table_v[pl.ds(i1 + h * 16, 16)]
                            s = plsc.bitcast(
                                plsc.bitcast(v0, jnp.bfloat16)
                                + plsc.bitcast(v1, jnp.bfloat16),
                                jnp.int32,
                            )
                            a = plsc.bitcast(s << 16, jnp.float32)
                            bvec = plsc.bitcast(s, jnp.float32)
                            bk = banks[cp % 2]
                            bk[2 * h] = bk[2 * h] + a
                            bk[2 * h + 1] = bk[2 * h + 1] + bvec
                    obase = (g * GROUP + w) * D
                    for k in range(4):
                        out_v[pl.ds(obase + k * 16, 16)] = (
                            acc[k] + ac2[k]
                        ) * (1.0 / CHARS)
                return carry2

            lax.fori_loop(0, CHUNK // GROUP, group_body, 0)

        def pair_body(p, carry):
            for b in range(2):
                ci = 2 * p + b
                if b == 0:
                    pltpu.async_copy(in_slice(ci + 1), idxb[1], sin[1])
                else:
                    @pl.when(p < n_pairs - 1)
                    def _():
                        pltpu.async_copy(in_slice(ci + 1), idxb[0], sin[0])

                pltpu.make_async_copy(in_slice(ci), idxb[b], sin[b]).wait()

                @pl.when(p > 0)
                def _():
                    pltpu.make_async_copy(
                        outb[b].at[pl.ds(0, CHUNK * D)],
                        out_slice(ci - 2),
                        son[b],
                    ).wait()

                compute_chunk(idxb[b], outb[b])
                pltpu.async_copy(
                    outb[b].at[pl.ds(0, CHUNK * D)], out_slice(ci), son[b]
                )
            return carry

        lax.fori_loop(0, n_pairs, pair_body, 0)
        for b in range(2):
            pltpu.make_async_copy(
                outb[b].at[pl.ds(0, CHUNK * D)],
                out_slice(n_chunks - 2 + b),
                son[b],
            ).wait()

    return sc_kernel


def kernel(x, table):
    b, w, c = x.shape
    n_words = b * w
    x_flat = x.reshape(n_words * c)
    table_p = jnp.pad(table, ((0, 0), (0, DP - D)))
    table_p = (
        table_p.reshape(VOCAB, 2, 2, 16)
        .transpose(0, 1, 3, 2)
        .astype(jnp.bfloat16)
        .reshape(VOCAB * DP // 2, 2)
    )
    table_p = lax.bitcast_convert_type(table_p, jnp.int32)
    out = _build_sc_call(n_words)(table_p, x_flat)
    return out.reshape(b, w, D)


# speedup vs baseline: 1.0159x; 1.0159x over previous
import functools

import jax
import jax.numpy as jnp
from jax import lax
from jax.experimental import pallas as pl
from jax.experimental.pallas import tpu as pltpu
from jax.experimental.pallas import tpu_sc as plsc

VOCAB = 1000
D = 50
DP = 64
CHARS = 20
NUM_CORES = 2
NUM_SUBCORES = 16
NTILES = NUM_CORES * NUM_SUBCORES
CHUNK = 640
GROUP = 8


def _build_sc_call(n_words):
    words_per_tile = n_words // NTILES
    n_chunks = words_per_tile // CHUNK
    n_pairs = n_chunks // 2
    mesh = plsc.VectorSubcoreMesh(core_axis_name="c", subcore_axis_name="s")

    @functools.partial(
        pl.kernel,
        out_type=jax.ShapeDtypeStruct((n_words * D,), jnp.float32),
        mesh=mesh,
        compiler_params=pltpu.CompilerParams(needs_layout_passes=False),
        scratch_types=[
            pltpu.VMEM((VOCAB * DP // 2,), jnp.int32),
            pltpu.VMEM((CHUNK * CHARS,), jnp.int32),
            pltpu.VMEM((CHUNK * CHARS,), jnp.int32),
            pltpu.VMEM((CHUNK * D + 16,), jnp.float32),
            pltpu.VMEM((CHUNK * D + 16,), jnp.float32),
            pltpu.SemaphoreType.DMA,
            pltpu.SemaphoreType.DMA,
            pltpu.SemaphoreType.DMA,
            pltpu.SemaphoreType.DMA,
        ],
    )
    def sc_kernel(
        table_hbm, x_hbm, out_hbm,
        table_v, idx0, idx1, out0, out1, si0, si1, so0, so1,
    ):
        wid = lax.axis_index("s") * NUM_CORES + lax.axis_index("c")
        tile_base = wid * words_per_tile
        idxb, outb = (idx0, idx1), (out0, out1)
        sin, son = (si0, si1), (so0, so1)

        def in_slice(ci):
            return x_hbm.at[
                pl.ds((tile_base + ci * CHUNK) * CHARS, CHUNK * CHARS)
            ]

        def out_slice(ci):
            return out_hbm.at[pl.ds((tile_base + ci * CHUNK) * D, CHUNK * D)]

        pltpu.async_copy(in_slice(0), idx0, si0)
        pltpu.sync_copy(table_hbm, table_v)

        def compute_chunk(idx_v, out_v):
            def group_body(g, carry2):
                base = g * (GROUP * CHARS)
                iv = [
                    idx_v[pl.ds(base + 16 * j, 16)] * (DP // 2)
                    for j in range(GROUP * CHARS // 16)
                ]
                for w in range(GROUP):
                    acc = [jnp.zeros((16,), jnp.float32) for _ in range(4)]
                    for cp in range(CHARS // 2):
                        f0 = w * CHARS + 2 * cp
                        i0 = iv[f0 // 16][f0 % 16]
                        i1 = iv[(f0 + 1) // 16][(f0 + 1) % 16]
                        for h in range(2):
                            v0 = table_v[pl.ds(i0 + h * 16, 16)]
                            v1 = table_v[pl.ds(i1 + h * 16, 16)]
                            s = plsc.bitcast(
                                plsc.bitcast(v0, jnp.bfloat16)
                                + plsc.bitcast(v1, jnp.bfloat16),
                                jnp.int32,
                            )
                            a = plsc.bitcast(s << 16, jnp.float32)
                            bvec = plsc.bitcast(s, jnp.float32)
                            acc[2 * h] = acc[2 * h] + a
                            acc[2 * h + 1] = acc[2 * h + 1] + bvec
                    obase = (g * GROUP + w) * D
                    for k in range(4):
                        out_v[pl.ds(obase + k * 16, 16)] = acc[k] * (1.0 / CHARS)
                return carry2

            lax.fori_loop(0, CHUNK // GROUP, group_body, 0)

        def pair_body(p, carry):
            for b in range(2):
                ci = 2 * p + b
                if b == 0:
                    pltpu.async_copy(in_slice(ci + 1), idxb[1], sin[1])
                else:
                    @pl.when(p < n_pairs - 1)
                    def _():
                        pltpu.async_copy(in_slice(ci + 1), idxb[0], sin[0])

                pltpu.make_async_copy(in_slice(ci), idxb[b], sin[b]).wait()

                @pl.when(p > 0)
                def _():
                    pltpu.make_async_copy(
                        outb[b].at[pl.ds(0, CHUNK * D)],
                        out_slice(ci - 2),
                        son[b],
                    ).wait()

                compute_chunk(idxb[b], outb[b])
                pltpu.async_copy(
                    outb[b].at[pl.ds(0, CHUNK * D)], out_slice(ci), son[b]
                )
            return carry

        lax.fori_loop(0, n_pairs, pair_body, 0)
        for b in range(2):
            pltpu.make_async_copy(
                outb[b].at[pl.ds(0, CHUNK * D)],
                out_slice(n_chunks - 2 + b),
                son[b],
            ).wait()

    return sc_kernel


def kernel(x, table):
    b, w, c = x.shape
    n_words = b * w
    x_flat = x.reshape(n_words * c)
    table_p = jnp.pad(table, ((0, 0), (0, DP - D)))
    table_p = (
        table_p.reshape(VOCAB, 2, 2, 16)
        .transpose(0, 1, 3, 2)
        .astype(jnp.bfloat16)
        .reshape(VOCAB * DP // 2, 2)
    )
    table_p = lax.bitcast_convert_type(table_p, jnp.int32)
    out = _build_sc_call(n_words)(table_p, x_flat)
    return out.reshape(b, w, D)
